# Initial kernel scaffold; baseline (speedup 1.0000x reference)
import jax
import jax.numpy as jnp
from jax.experimental import pallas as pl


def kernel(x, edge_index, batch, W1, b1, g1, be1, W2, b2, g2, be2, W3, b3, g3, be3, Wfc, bfc):
    raise NotImplementedError("write your pallas kernel here")


# speedup vs baseline: 2.2123x; 2.2123x over previous
import functools

import jax
import jax.numpy as jnp
from jax.experimental import pallas as pl
from jax.experimental.pallas import tpu as pltpu

N_NODES = 100000
N_GRAPHS = 64
EPS = 1e-5
BM = 1000


def _mm_stats_body(e_ref, hs_ref, dinv_ref, w_ref, b_ref, y_ref, s1_ref, s2_ref):
    agg = dinv_ref[...] * (e_ref[...] + hs_ref[...])
    y = jnp.dot(agg, w_ref[...], preferred_element_type=jnp.float32) + b_ref[...]
    y_ref[...] = y
    s1_ref[...] = jnp.sum(y, axis=0, keepdims=True)[None]
    s2_ref[...] = jnp.sum(y * y, axis=0, keepdims=True)[None]


def _mm_stats(e, hs, dinv, w, b):
    n, k = e.shape
    dout = w.shape[1]
    nb = n // BM
    y, s1, s2 = pl.pallas_call(
        _mm_stats_body,
        grid=(nb,),
        in_specs=[
            pl.BlockSpec((BM, k), lambda i: (i, 0)),
            pl.BlockSpec((BM, k), lambda i: (i, 0)),
            pl.BlockSpec((BM, 1), lambda i: (i, 0)),
            pl.BlockSpec((k, dout), lambda i: (0, 0)),
            pl.BlockSpec((1, dout), lambda i: (0, 0)),
        ],
        out_specs=[
            pl.BlockSpec((BM, dout), lambda i: (i, 0)),
            pl.BlockSpec((1, 1, dout), lambda i: (i, 0, 0)),
            pl.BlockSpec((1, 1, dout), lambda i: (i, 0, 0)),
        ],
        out_shape=[
            jax.ShapeDtypeStruct((n, dout), jnp.float32),
            jax.ShapeDtypeStruct((nb, 1, dout), jnp.float32),
            jax.ShapeDtypeStruct((nb, 1, dout), jnp.float32),
        ],
    )(e, hs, dinv, w, b.reshape(1, dout))
    return y, s1, s2


def _affine_relu_body(y_ref, a_ref, c_ref, dinv_ref, o_ref):
    o_ref[...] = jax.nn.relu(y_ref[...] * a_ref[...] + c_ref[...]) * dinv_ref[...]


def _affine_relu_scale(y, a, c, dinv):
    n, d = y.shape
    return pl.pallas_call(
        _affine_relu_body,
        grid=(n // BM,),
        in_specs=[
            pl.BlockSpec((BM, d), lambda i: (i, 0)),
            pl.BlockSpec((1, d), lambda i: (0, 0)),
            pl.BlockSpec((1, d), lambda i: (0, 0)),
            pl.BlockSpec((BM, 1), lambda i: (i, 0)),
        ],
        out_specs=pl.BlockSpec((BM, d), lambda i: (i, 0)),
        out_shape=jax.ShapeDtypeStruct((n, d), jnp.float32),
    )(y, a.reshape(1, d), c.reshape(1, d), dinv)


def _fc_body(p_ref, w_ref, b_ref, o_ref):
    o_ref[...] = jax.nn.relu(
        jnp.dot(p_ref[...], w_ref[...], preferred_element_type=jnp.float32)
        + b_ref[...]
    )


def _bn_affine(s1, s2, g, be, n):
    mu = jnp.sum(s1, axis=(0, 1)) / n
    var = jnp.sum(s2, axis=(0, 1)) / n - mu * mu
    a = g * jax.lax.rsqrt(var + EPS)
    c = be - mu * a
    return a, c


def kernel(x, edge_index, batch, W1, b1, g1, be1, W2, b2, g2, be2, W3, b3,
           g3, be3, Wfc, bfc):
    n = x.shape[0]
    src, dst = edge_index[0], edge_index[1]

    deg = jax.ops.segment_sum(jnp.ones(src.shape, jnp.float32), dst,
                              num_segments=n) + 1.0
    dinv = jax.lax.rsqrt(deg)[:, None]

    def agg_xla(hs):
        return jax.ops.segment_sum(hs[src], dst, num_segments=n)

    xs = x * dinv
    e1 = agg_xla(xs)
    pad = jnp.zeros((n, 2), jnp.float32)
    e1p = jnp.concatenate([e1, pad], axis=1)
    xsp = jnp.concatenate([xs, pad], axis=1)
    W1p = jnp.concatenate([W1, jnp.zeros((2, W1.shape[1]), jnp.float32)], axis=0)
    y1, s1a, s1b = _mm_stats(e1p, xsp, dinv, W1p, b1)
    a1, c1 = _bn_affine(s1a, s1b, g1, be1, n)
    hs1 = _affine_relu_scale(y1, a1, c1, dinv)

    e2 = agg_xla(hs1)
    y2, s2a, s2b = _mm_stats(e2, hs1, dinv, W2, b2)
    a2, c2 = _bn_affine(s2a, s2b, g2, be2, n)
    hs2 = _affine_relu_scale(y2, a2, c2, dinv)

    e3 = agg_xla(hs2)
    y3, s3a, s3b = _mm_stats(e3, hs2, dinv, W3, b3)
    a3, c3 = _bn_affine(s3a, s3b, g3, be3, n)
    h3 = _affine_relu_scale(y3, a3, c3, jnp.ones_like(dinv))

    sums = jax.ops.segment_sum(h3, batch, num_segments=N_GRAPHS)
    cnt = jax.ops.segment_sum(jnp.ones((n,), jnp.float32), batch,
                              num_segments=N_GRAPHS)
    pooled = sums / jnp.maximum(cnt, 1.0)[:, None]

    out = pl.pallas_call(
        _fc_body,
        in_specs=[
            pl.BlockSpec(pooled.shape, lambda: (0, 0)),
            pl.BlockSpec(Wfc.shape, lambda: (0, 0)),
            pl.BlockSpec((1, bfc.shape[0]), lambda: (0, 0)),
        ],
        out_specs=pl.BlockSpec((N_GRAPHS, bfc.shape[0]), lambda: (0, 0)),
        out_shape=jax.ShapeDtypeStruct((N_GRAPHS, bfc.shape[0]), jnp.float32),
    )(pooled, Wfc, bfc.reshape(1, -1))
    return out
